# baseline (speedup 1.0000x reference)
import functools

import jax
import jax.numpy as jnp
from jax import lax
from jax.experimental import pallas as pl
from jax.experimental.pallas import tpu as pltpu
from jax.experimental.pallas import tpu_sc as plsc

N = 10000
E = 160000
D = 16
DD = 256
NCORE = 2
NSUB = 16
NW = NCORE * NSUB
RPF = DD // NW
EHB = E // 128
CHB = 5
CH = CHB * 128
NCH = EHB // CHB
NBUF = 2
NSTEP = NCH // NBUF


def _sc_corr_kernel(st, ct, dst_hbm, out_hbm, sbuf, cbuf, ibuf, acc,
                    gsem0, gsem1):
    cid = lax.axis_index("c")
    sid = lax.axis_index("s")
    wid = sid * NCORE + cid
    r0 = wid * RPF
    dmaj = wid // 2
    ehi = wid % 2
    gsem = (gsem0, gsem1)

    def gather(t, b):
        return (
            pltpu.make_async_copy(
                st.at[dmaj, ehi, pl.ds(t * CHB, CHB)], sbuf.at[b], gsem[b]),
            pltpu.make_async_copy(
                ct.at[dmaj, ehi, pl.ds(t * CHB, CHB)], cbuf.at[b], gsem[b]),
            pltpu.make_async_copy(
                dst_hbm.at[pl.ds(t * CH, CH)], ibuf.at[b], gsem[b]),
        )

    for b in range(NBUF):
        for cp in gather(b, b):
            cp.start()

    zero = jnp.zeros((16,), jnp.float32)

    @plsc.parallel_loop(0, N // 16, 1, unroll=4)
    def _z(i):
        for r in range(RPF):
            acc[r, pl.ds(i * 16, 16)] = zero

    def _step(stp, _):
        for b in range(NBUF):
            t = stp * NBUF + b
            for cp in gather(t, b):
                cp.wait()

            @plsc.parallel_loop(0, CHB, 1, unroll=5)
            def _grp(i):
                for g in range(8):
                    idx = ibuf[b, pl.ds(i * 128 + g * 16, 16)]
                    lsl = pl.ds(g * 16, 16)
                    for r in range(RPF):
                        p = sbuf[b, i, r, lsl] * cbuf[b, i, r, lsl]
                        plsc.addupdate_scatter(acc.at[r], [idx], p)

            @pl.when(t + NBUF < NCH)
            def _():
                for cp in gather(t + NBUF, b):
                    cp.start()
        return 0

    lax.fori_loop(0, NSTEP, _step, 0)

    pltpu.sync_copy(acc, out_hbm.at[pl.ds(r0, RPF)])


@functools.cache
def _sc_corr():
    return functools.partial(
        pl.kernel,
        mesh=plsc.VectorSubcoreMesh(core_axis_name="c", subcore_axis_name="s"),
        out_type=jax.ShapeDtypeStruct((DD, N), jnp.float32),
        scratch_types=[
            pltpu.VMEM((NBUF, CHB, RPF, 128), jnp.float32),
            pltpu.VMEM((NBUF, CHB, RPF, 128), jnp.float32),
            pltpu.VMEM((NBUF, CH), jnp.int32),
            pltpu.VMEM((RPF, N), jnp.float32),
            pltpu.SemaphoreType.DMA,
            pltpu.SemaphoreType.DMA,
        ],
        compiler_params=pltpu.CompilerParams(use_tc_tiling_on_sc=False,
                                             needs_layout_passes=False),
    )(_sc_corr_kernel)


def _tc_tail_kernel(corrt_ref, m_ref, bd1_ref, b1_ref, bd2_ref, b2_ref,
                    wft_ref, biast_ref, out_ref):
    ctr = corrt_ref[...]
    scores = lax.dot_general(m_ref[...], ctr,
                             (((1,), (0,)), ((), ())),
                             preferred_element_type=jnp.float32)
    s = scores - jnp.max(scores)
    e = jnp.exp(s)
    att = e / jnp.sum(e)
    ci = lax.dot_general(att, ctr, (((1,), (1,)), ((), ())),
                         preferred_element_type=jnp.float32)
    fi = lax.dot_general(ci, bd1_ref[...], (((1,), (0,)), ((), ())),
                         preferred_element_type=jnp.float32)
    fi = jnp.maximum(fi + b1_ref[...], 0.0)
    fi = 1.0 / (1.0 + jnp.exp(-fi))
    fi2 = lax.dot_general(fi, bd2_ref[...], (((1,), (0,)), ((), ())),
                          preferred_element_type=jnp.float32)
    fi2 = jnp.maximum(fi2 + b2_ref[...], 0.0)
    out = lax.dot_general(fi2, wft_ref[...], (((1,), (0,)), ((), ())),
                          preferred_element_type=jnp.float32)
    out_ref[...] = out + biast_ref[...]


def kernel(Sij, Cijj, dst, Wc, alphaC, alphaf, b, w, Wf, bias, W1, b1, W2, b2):
    def _tile_view(x):
        return (x.transpose(1, 2, 0)
                 .reshape(D, 2, 8, EHB, 128)
                 .transpose(0, 1, 3, 2, 4))

    corr_t = _sc_corr()(_tile_view(Sij), _tile_view(Cijj), dst)

    u = (w @ Wc).reshape(D)
    v = alphaC.reshape(D)
    m = (u[:, None] * v[None, :]).reshape(1, DD)
    eye_d = jnp.eye(D, dtype=jnp.float32)
    bd1 = (eye_d[:, None, :, None] * W1.T[None, :, None, :]).reshape(DD, D * 256)
    b1big = jnp.tile(b1, D).reshape(1, D * 256)
    eye_f = jnp.eye(256, dtype=jnp.float32)
    bd2 = (W2.reshape(D, 1, 1) * eye_f[None, :, :]).reshape(D * 256, 256)
    b2big = jnp.full((1, 256), b2[0], dtype=jnp.float32)

    P = pl.pallas_call(
        _tc_tail_kernel,
        out_shape=jax.ShapeDtypeStruct((1, D), jnp.float32),
    )(corr_t, m, bd1, b1big, bd2, b2big, Wf.T, bias.T)
    return P


# speedup vs baseline: 1.0783x; 1.0783x over previous
import functools

import jax
import jax.numpy as jnp
from jax import lax
from jax.experimental import pallas as pl
from jax.experimental.pallas import tpu as pltpu
from jax.experimental.pallas import tpu_sc as plsc

N = 10000
E = 160000
D = 16
DD = 256
NCORE = 2
NSUB = 16
NW = NCORE * NSUB
RPF = DD // NW
EHB = E // 128
CHB = 5
CH = CHB * 128
NCH = EHB // CHB
NBUF = 2
NSTEP = NCH // NBUF


def _sc_corr_kernel(st, ct, dst_hbm, out_hbm, sbuf, cbuf, ibuf, acc,
                    gsem0, gsem1):
    cid = lax.axis_index("c")
    sid = lax.axis_index("s")
    wid = sid * NCORE + cid
    r0 = wid * RPF
    dmaj = wid // 2
    ehi = wid % 2
    gsem = (gsem0, gsem1)

    def gather(t, b):
        return (
            pltpu.make_async_copy(
                st.at[dmaj, ehi, pl.ds(t * CHB, CHB)], sbuf.at[b], gsem[b]),
            pltpu.make_async_copy(
                ct.at[dmaj, ehi, pl.ds(t * CHB, CHB)], cbuf.at[b], gsem[b]),
            pltpu.make_async_copy(
                dst_hbm.at[pl.ds(t * CH, CH)], ibuf.at[b], gsem[b]),
        )

    for b in range(NBUF):
        for cp in gather(b, b):
            cp.start()

    zero = jnp.zeros((16,), jnp.float32)

    @plsc.parallel_loop(0, N // 16, 1, unroll=4)
    def _z(i):
        for r in range(RPF):
            acc[r, pl.ds(i * 16, 16)] = zero

    def _step(stp, _):
        for b in range(NBUF):
            t = stp * NBUF + b
            for cp in gather(t, b):
                cp.wait()

            @plsc.parallel_loop(0, CHB, 1, unroll=2)
            def _grp(i):
                for g in range(8):
                    idx = ibuf[b, pl.ds(i * 128 + g * 16, 16)]
                    lsl = pl.ds(g * 16, 16)
                    for r in range(RPF):
                        p = sbuf[b, i, r, lsl] * cbuf[b, i, r, lsl]
                        plsc.addupdate_scatter(acc.at[r], [idx], p)

            @pl.when(t + NBUF < NCH)
            def _():
                for cp in gather(t + NBUF, b):
                    cp.start()
        return 0

    lax.fori_loop(0, NSTEP, _step, 0)

    pltpu.sync_copy(acc, out_hbm.at[pl.ds(r0, RPF)])


@functools.cache
def _sc_corr():
    return functools.partial(
        pl.kernel,
        mesh=plsc.VectorSubcoreMesh(core_axis_name="c", subcore_axis_name="s"),
        out_type=jax.ShapeDtypeStruct((DD, N), jnp.float32),
        scratch_types=[
            pltpu.VMEM((NBUF, CHB, RPF, 128), jnp.float32),
            pltpu.VMEM((NBUF, CHB, RPF, 128), jnp.float32),
            pltpu.VMEM((NBUF, CH), jnp.int32),
            pltpu.VMEM((RPF, N), jnp.float32),
            pltpu.SemaphoreType.DMA,
            pltpu.SemaphoreType.DMA,
        ],
        compiler_params=pltpu.CompilerParams(use_tc_tiling_on_sc=False,
                                             needs_layout_passes=False),
    )(_sc_corr_kernel)


def _tc_tail_kernel(corrt_ref, m_ref, bd1_ref, b1_ref, bd2_ref, b2_ref,
                    wft_ref, biast_ref, out_ref):
    ctr = corrt_ref[...]
    scores = lax.dot_general(m_ref[...], ctr,
                             (((1,), (0,)), ((), ())),
                             preferred_element_type=jnp.float32)
    s = scores - jnp.max(scores)
    e = jnp.exp(s)
    att = e / jnp.sum(e)
    ci = lax.dot_general(att, ctr, (((1,), (1,)), ((), ())),
                         preferred_element_type=jnp.float32)
    fi = lax.dot_general(ci, bd1_ref[...], (((1,), (0,)), ((), ())),
                         preferred_element_type=jnp.float32)
    fi = jnp.maximum(fi + b1_ref[...], 0.0)
    fi = 1.0 / (1.0 + jnp.exp(-fi))
    fi2 = lax.dot_general(fi, bd2_ref[...], (((1,), (0,)), ((), ())),
                          preferred_element_type=jnp.float32)
    fi2 = jnp.maximum(fi2 + b2_ref[...], 0.0)
    out = lax.dot_general(fi2, wft_ref[...], (((1,), (0,)), ((), ())),
                          preferred_element_type=jnp.float32)
    out_ref[...] = out + biast_ref[...]


def kernel(Sij, Cijj, dst, Wc, alphaC, alphaf, b, w, Wf, bias, W1, b1, W2, b2):
    def _tile_view(x):
        return (x.transpose(1, 2, 0)
                 .reshape(D, 2, 8, EHB, 128)
                 .transpose(0, 1, 3, 2, 4))

    corr_t = _sc_corr()(_tile_view(Sij), _tile_view(Cijj), dst)

    u = (w @ Wc).reshape(D)
    v = alphaC.reshape(D)
    m = (u[:, None] * v[None, :]).reshape(1, DD)
    eye_d = jnp.eye(D, dtype=jnp.float32)
    bd1 = (eye_d[:, None, :, None] * W1.T[None, :, None, :]).reshape(DD, D * 256)
    b1big = jnp.tile(b1, D).reshape(1, D * 256)
    eye_f = jnp.eye(256, dtype=jnp.float32)
    bd2 = (W2.reshape(D, 1, 1) * eye_f[None, :, :]).reshape(D * 256, 256)
    b2big = jnp.full((1, 256), b2[0], dtype=jnp.float32)

    P = pl.pallas_call(
        _tc_tail_kernel,
        out_shape=jax.ShapeDtypeStruct((1, D), jnp.float32),
    )(corr_t, m, bd1, b1big, bd2, b2big, Wf.T, bias.T)
    return P


# speedup vs baseline: 1.3331x; 1.2363x over previous
import functools

import jax
import jax.numpy as jnp
from jax import lax
from jax.experimental import pallas as pl
from jax.experimental.pallas import tpu as pltpu
from jax.experimental.pallas import tpu_sc as plsc

N = 10000
E = 160000
D = 16
DD = 256
NCORE = 2
NSUB = 16
NW = NCORE * NSUB
RPF = DD // NW
EHB = E // 128
CHB = 5
CH = CHB * 128
NCH = EHB // CHB
NBUF = 2
NSTEP = NCH // NBUF


def _sc_corr_kernel(st, ct, dst_hbm, out_hbm, sbuf, cbuf, ibuf, acc,
                    gsem0, gsem1):
    cid = lax.axis_index("c")
    sid = lax.axis_index("s")
    wid = sid * NCORE + cid
    r0 = wid * RPF
    dmaj = wid // 2
    ehi = wid % 2
    gsem = (gsem0, gsem1)

    def gather(t, b):
        return (
            pltpu.make_async_copy(
                st.at[dmaj, ehi, pl.ds(t * CHB, CHB)], sbuf.at[b], gsem[b]),
            pltpu.make_async_copy(
                ct.at[dmaj, ehi, pl.ds(t * CHB, CHB)], cbuf.at[b], gsem[b]),
            pltpu.make_async_copy(
                dst_hbm.at[pl.ds(t * CH, CH)], ibuf.at[b], gsem[b]),
        )

    for b in range(NBUF):
        for cp in gather(b, b):
            cp.start()

    zero = jnp.zeros((16,), jnp.float32)

    def _z(i, _):
        for r in range(RPF):
            acc[r, pl.ds(i * 16, 16)] = zero
        return 0

    lax.fori_loop(0, N // 16, _z, 0)

    def _step(stp, _):
        for b in range(NBUF):
            t = stp * NBUF + b
            for cp in gather(t, b):
                cp.wait()

            @plsc.parallel_loop(0, CHB, 1, unroll=2)
            def _grp(i):
                for g in range(8):
                    idx = ibuf[b, pl.ds(i * 128 + g * 16, 16)]
                    lsl = pl.ds(g * 16, 16)
                    for r in range(RPF):
                        p = sbuf[b, i, r, lsl] * cbuf[b, i, r, lsl]
                        plsc.addupdate_scatter(acc.at[r], [idx], p)

            @pl.when(t + NBUF < NCH)
            def _():
                for cp in gather(t + NBUF, b):
                    cp.start()
        return 0

    lax.fori_loop(0, NSTEP, _step, 0)

    pltpu.sync_copy(acc, out_hbm.at[pl.ds(r0, RPF)])


@functools.cache
def _sc_corr():
    return functools.partial(
        pl.kernel,
        mesh=plsc.VectorSubcoreMesh(core_axis_name="c", subcore_axis_name="s"),
        out_type=jax.ShapeDtypeStruct((DD, N), jnp.float32),
        scratch_types=[
            pltpu.VMEM((NBUF, CHB, RPF, 128), jnp.float32),
            pltpu.VMEM((NBUF, CHB, RPF, 128), jnp.float32),
            pltpu.VMEM((NBUF, CH), jnp.int32),
            pltpu.VMEM((RPF, N), jnp.float32),
            pltpu.SemaphoreType.DMA,
            pltpu.SemaphoreType.DMA,
        ],
        compiler_params=pltpu.CompilerParams(use_tc_tiling_on_sc=False,
                                             needs_layout_passes=False),
    )(_sc_corr_kernel)


def _tc_tail_kernel(corrt_ref, m_ref, bd1_ref, b1_ref, bd2_ref, b2_ref,
                    wft_ref, biast_ref, out_ref):
    ctr = corrt_ref[...]
    scores = lax.dot_general(m_ref[...], ctr,
                             (((1,), (0,)), ((), ())),
                             preferred_element_type=jnp.float32)
    s = scores - jnp.max(scores)
    e = jnp.exp(s)
    att = e / jnp.sum(e)
    ci = lax.dot_general(att, ctr, (((1,), (1,)), ((), ())),
                         preferred_element_type=jnp.float32)
    fi = lax.dot_general(ci, bd1_ref[...], (((1,), (0,)), ((), ())),
                         preferred_element_type=jnp.float32)
    fi = jnp.maximum(fi + b1_ref[...], 0.0)
    fi = 1.0 / (1.0 + jnp.exp(-fi))
    fi2 = lax.dot_general(fi, bd2_ref[...], (((1,), (0,)), ((), ())),
                          preferred_element_type=jnp.float32)
    fi2 = jnp.maximum(fi2 + b2_ref[...], 0.0)
    out = lax.dot_general(fi2, wft_ref[...], (((1,), (0,)), ((), ())),
                          preferred_element_type=jnp.float32)
    out_ref[...] = out + biast_ref[...]


def kernel(Sij, Cijj, dst, Wc, alphaC, alphaf, b, w, Wf, bias, W1, b1, W2, b2):
    def _tile_view(x):
        return (x.transpose(1, 2, 0)
                 .reshape(D, 2, 8, EHB, 128)
                 .transpose(0, 1, 3, 2, 4))

    corr_t = _sc_corr()(_tile_view(Sij), _tile_view(Cijj), dst)

    u = (w @ Wc).reshape(D)
    v = alphaC.reshape(D)
    m = (u[:, None] * v[None, :]).reshape(1, DD)
    eye_d = jnp.eye(D, dtype=jnp.float32)
    bd1 = (eye_d[:, None, :, None] * W1.T[None, :, None, :]).reshape(DD, D * 256)
    b1big = jnp.tile(b1, D).reshape(1, D * 256)
    eye_f = jnp.eye(256, dtype=jnp.float32)
    bd2 = (W2.reshape(D, 1, 1) * eye_f[None, :, :]).reshape(D * 256, 256)
    b2big = jnp.full((1, 256), b2[0], dtype=jnp.float32)

    P = pl.pallas_call(
        _tc_tail_kernel,
        out_shape=jax.ShapeDtypeStruct((1, D), jnp.float32),
    )(corr_t, m, bd1, b1big, bd2, b2big, Wf.T, bias.T)
    return P


# speedup vs baseline: 1.6559x; 1.2422x over previous
import functools

import jax
import jax.numpy as jnp
from jax import lax
from jax.experimental import pallas as pl
from jax.experimental.pallas import tpu as pltpu
from jax.experimental.pallas import tpu_sc as plsc

N = 10000
E = 160000
D = 16
DD = 256
NCORE = 2
NSUB = 16
NW = NCORE * NSUB
RPF = DD // NW
EHB = E // 128
CHB = 5
CH = CHB * 128
NCH = EHB // CHB
NBUF = 2
NSTEP = NCH // NBUF


def _sc_corr_kernel(st, ct, dst_hbm, out_hbm, sbuf, cbuf, ibuf, acc,
                    gsem0, gsem1):
    cid = lax.axis_index("c")
    sid = lax.axis_index("s")
    wid = sid * NCORE + cid
    r0 = wid * RPF
    dmaj = wid // 2
    ehi = wid % 2
    gsem = (gsem0, gsem1)

    def gather(t, b):
        return (
            pltpu.make_async_copy(
                st.at[dmaj, ehi, pl.ds(t * CHB, CHB)], sbuf.at[b], gsem[b]),
            pltpu.make_async_copy(
                ct.at[dmaj, ehi, pl.ds(t * CHB, CHB)], cbuf.at[b], gsem[b]),
            pltpu.make_async_copy(
                dst_hbm.at[pl.ds(t * CH, CH)], ibuf.at[b], gsem[b]),
        )

    for b in range(NBUF):
        for cp in gather(b, b):
            cp.start()

    zero = jnp.zeros((16,), jnp.float32)

    def _z(i, _):
        for r in range(RPF):
            acc[r, pl.ds(i * 16, 16)] = zero
        return 0

    lax.fori_loop(0, N // 16, _z, 0)

    def _step(stp, _):
        for b in range(NBUF):
            t = stp * NBUF + b
            for cp in gather(t, b):
                cp.wait()

            @plsc.parallel_loop(0, CHB * 8, 1, unroll=4)
            def _grp(j):
                ib = j // 8
                g = j % 8
                idx = ibuf[b, pl.ds(j * 16, 16)]
                lsl = pl.ds(g * 16, 16)
                for r in range(RPF):
                    p = sbuf[b, ib, r, lsl] * cbuf[b, ib, r, lsl]
                    plsc.addupdate_scatter(acc.at[r], [idx], p)

            @pl.when(t + NBUF < NCH)
            def _():
                for cp in gather(t + NBUF, b):
                    cp.start()
        return 0

    lax.fori_loop(0, NSTEP, _step, 0)

    pltpu.sync_copy(acc, out_hbm.at[pl.ds(r0, RPF)])


@functools.cache
def _sc_corr():
    return functools.partial(
        pl.kernel,
        mesh=plsc.VectorSubcoreMesh(core_axis_name="c", subcore_axis_name="s"),
        out_type=jax.ShapeDtypeStruct((DD, N), jnp.float32),
        scratch_types=[
            pltpu.VMEM((NBUF, CHB, RPF, 128), jnp.float32),
            pltpu.VMEM((NBUF, CHB, RPF, 128), jnp.float32),
            pltpu.VMEM((NBUF, CH), jnp.int32),
            pltpu.VMEM((RPF, N), jnp.float32),
            pltpu.SemaphoreType.DMA,
            pltpu.SemaphoreType.DMA,
        ],
        compiler_params=pltpu.CompilerParams(use_tc_tiling_on_sc=False,
                                             needs_layout_passes=False),
    )(_sc_corr_kernel)


def _tc_tail_kernel(corrt_ref, m_ref, bd1_ref, b1_ref, bd2_ref, b2_ref,
                    wft_ref, biast_ref, out_ref):
    ctr = corrt_ref[...]
    scores = lax.dot_general(m_ref[...], ctr,
                             (((1,), (0,)), ((), ())),
                             preferred_element_type=jnp.float32)
    s = scores - jnp.max(scores)
    e = jnp.exp(s)
    att = e / jnp.sum(e)
    ci = lax.dot_general(att, ctr, (((1,), (1,)), ((), ())),
                         preferred_element_type=jnp.float32)
    fi = lax.dot_general(ci, bd1_ref[...], (((1,), (0,)), ((), ())),
                         preferred_element_type=jnp.float32)
    fi = jnp.maximum(fi + b1_ref[...], 0.0)
    fi = 1.0 / (1.0 + jnp.exp(-fi))
    fi2 = lax.dot_general(fi, bd2_ref[...], (((1,), (0,)), ((), ())),
                          preferred_element_type=jnp.float32)
    fi2 = jnp.maximum(fi2 + b2_ref[...], 0.0)
    out = lax.dot_general(fi2, wft_ref[...], (((1,), (0,)), ((), ())),
                          preferred_element_type=jnp.float32)
    out_ref[...] = out + biast_ref[...]


def kernel(Sij, Cijj, dst, Wc, alphaC, alphaf, b, w, Wf, bias, W1, b1, W2, b2):
    def _tile_view(x):
        return (x.transpose(1, 2, 0)
                 .reshape(D, 2, 8, EHB, 128)
                 .transpose(0, 1, 3, 2, 4))

    corr_t = _sc_corr()(_tile_view(Sij), _tile_view(Cijj), dst)

    u = (w @ Wc).reshape(D)
    v = alphaC.reshape(D)
    m = (u[:, None] * v[None, :]).reshape(1, DD)
    eye_d = jnp.eye(D, dtype=jnp.float32)
    bd1 = (eye_d[:, None, :, None] * W1.T[None, :, None, :]).reshape(DD, D * 256)
    b1big = jnp.tile(b1, D).reshape(1, D * 256)
    eye_f = jnp.eye(256, dtype=jnp.float32)
    bd2 = (W2.reshape(D, 1, 1) * eye_f[None, :, :]).reshape(D * 256, 256)
    b2big = jnp.full((1, 256), b2[0], dtype=jnp.float32)

    P = pl.pallas_call(
        _tc_tail_kernel,
        out_shape=jax.ShapeDtypeStruct((1, D), jnp.float32),
    )(corr_t, m, bd1, b1big, bd2, b2big, Wf.T, bias.T)
    return P
